# baseline (speedup 1.0000x reference)
import math

import jax
import jax.numpy as jnp
from jax.experimental import pallas as pl
from jax.experimental.pallas import tpu as pltpu

_SCALE = 400.0 / math.log(10.0)
_DEFAULT_PRED = 7.6699353278706015
_NEG = -1e30

_TOTAL = 32768
_D = 256
_H = 32
_B = 16
_BLK = 8704
_GRID = -(-_TOTAL // _BLK)


def _fused_kernel(x_ref, w1t_ref, wrz_ref, meta_ref, out_ref, acc_ref):
    i = pl.program_id(0)
    nprog = pl.num_programs(0)

    @pl.when(i == 0)
    def _init():
        acc_ref[:, 0:1] = jnp.full((_B, 1), _NEG, jnp.float32)
        acc_ref[:, 1:2] = jnp.zeros((_B, 1), jnp.float32)
        acc_ref[:, 2:3] = jnp.zeros((_B, 1), jnp.float32)

    xb = x_ref[:]
    b1 = meta_ref[0:1, 0:_H]
    brz = meta_ref[1:2, 0:2]
    starts = meta_ref[2:3, 0:_B].reshape(_B, 1)
    ends = meta_ref[3:4, 0:_B].reshape(_B, 1)
    n_used = meta_ref[4:5, 0:1]

    hb = jnp.maximum(
        jnp.dot(
            xb.astype(jnp.bfloat16), w1t_ref[:].astype(jnp.bfloat16),
            preferred_element_type=jnp.float32,
        ) + b1,
        0.0,
    )
    rz = jnp.dot(hb, wrz_ref[:], preferred_element_type=jnp.float32) + brz
    rzt = rz.T

    idx = (
        jax.lax.broadcasted_iota(jnp.int32, (1, _BLK), 1) + i * _BLK
    ).astype(jnp.float32)
    in_used = idx < n_used
    r = jnp.where(in_used, rzt[0:1, :], 0.0)
    z = jnp.where(in_used, rzt[1:2, :], _NEG)

    mask = (idx >= starts) & (idx < ends)
    zm = jnp.where(mask, z, _NEG)

    old_max = acc_ref[:, 0:1]
    blk_max = jnp.max(zm, axis=1, keepdims=True)
    new_max = jnp.maximum(old_max, blk_max)
    scale = jnp.exp(old_max - new_max)

    e = jnp.exp(zm - new_max) * mask.astype(jnp.float32)
    s = jnp.sum(e, axis=1, keepdims=True)
    sr = jnp.sum(e * r, axis=1, keepdims=True)

    acc_ref[:, 0:1] = new_max
    acc_ref[:, 1:2] = acc_ref[:, 1:2] * scale + s
    acc_ref[:, 2:3] = acc_ref[:, 2:3] * scale + sr

    @pl.when(i == nprog - 1)
    def _finish():
        denom = acc_ref[:, 1:2]
        preds = acc_ref[:, 2:3] / jnp.where(denom == 0.0, 1.0, denom)
        empty = starts == ends
        preds = jnp.where(empty, _DEFAULT_PRED, preds)
        out_ref[:] = _SCALE * preds


def kernel(x, xlens, W1, b1, Wr, br, Wz, bz):
    w1t = W1.T
    wrz = jnp.concatenate([Wr, Wz], axis=0).T

    clens = jnp.concatenate(
        [jnp.zeros((1,), dtype=xlens.dtype), jnp.cumsum(xlens)]
    )
    starts = clens[:-1].astype(jnp.float32)
    ends = clens[1:].astype(jnp.float32)
    n_used = jnp.minimum(clens[-1], _TOTAL).astype(jnp.int32)
    nblocks = jnp.clip((n_used + _BLK - 1) // _BLK, 1, _GRID)

    meta = jnp.zeros((8, _H), jnp.float32)
    meta = meta.at[0, :].set(b1)
    meta = meta.at[1, 0].set(br[0])
    meta = meta.at[1, 1].set(bz[0])
    meta = meta.at[2, :_B].set(starts)
    meta = meta.at[3, :_B].set(ends)
    meta = meta.at[4, 0].set(n_used.astype(jnp.float32))

    out = pl.pallas_call(
        _fused_kernel,
        grid=(nblocks,),
        in_specs=[
            pl.BlockSpec((_BLK, _D), lambda i: (i, 0)),
            pl.BlockSpec((_D, _H), lambda i: (0, 0)),
            pl.BlockSpec((_H, 2), lambda i: (0, 0)),
            pl.BlockSpec((8, _H), lambda i: (0, 0)),
        ],
        out_specs=pl.BlockSpec((_B, 1), lambda i: (0, 0)),
        out_shape=jax.ShapeDtypeStruct((_B, 1), jnp.float32),
        scratch_shapes=[pltpu.VMEM((_B, 8), jnp.float32)],
    )(x, w1t, wrz, meta)
    return out.reshape(_B)


# speedup vs baseline: 1.2605x; 1.2605x over previous
import math

import jax
import jax.numpy as jnp
from jax.experimental import pallas as pl
from jax.experimental.pallas import tpu as pltpu

_SCALE = 400.0 / math.log(10.0)
_DEFAULT_PRED = 7.6699353278706015
_NEG = -1e30

_TOTAL = 32768
_D = 256
_H = 32
_B = 16
_BLK = 6400
_GRID = -(-_TOTAL // _BLK)


def _fused_kernel(x_ref, w1_ref, wr_ref, wz_ref, b1_ref, br_ref, bz_ref,
                  xlens_ref, out_ref, acc_ref):
    i = pl.program_id(0)
    nprog = pl.num_programs(0)

    @pl.when(i == 0)
    def _init():
        acc_ref[:, 0:1] = jnp.full((_B, 1), _NEG, jnp.float32)
        acc_ref[:, 1:2] = jnp.zeros((_B, 1), jnp.float32)
        acc_ref[:, 2:3] = jnp.zeros((_B, 1), jnp.float32)

    xl = xlens_ref[0:1, :].astype(jnp.float32)
    row = jax.lax.broadcasted_iota(jnp.int32, (_B, _B), 0)
    col = jax.lax.broadcasted_iota(jnp.int32, (_B, _B), 1)
    ends = jnp.sum(jnp.where(col <= row, xl, 0.0), axis=1, keepdims=True)
    starts = ends - jnp.sum(
        jnp.where(col == row, xl, 0.0), axis=1, keepdims=True
    )
    n_used = jnp.minimum(jnp.max(ends), float(_TOTAL))

    xb = x_ref[:]
    hb = jnp.maximum(
        jax.lax.dot_general(
            xb.astype(jnp.bfloat16), w1_ref[:].astype(jnp.bfloat16),
            (((1,), (1,)), ((), ())),
            preferred_element_type=jnp.float32,
        ) + b1_ref[0:1, :],
        0.0,
    )
    wrz = jnp.concatenate([wr_ref[0:1, :], wz_ref[0:1, :]], axis=0)
    rz = jax.lax.dot_general(
        hb, wrz, (((1,), (1,)), ((), ())),
        preferred_element_type=jnp.float32,
    )
    rzt = rz.T

    idx = (
        jax.lax.broadcasted_iota(jnp.int32, (1, _BLK), 1) + i * _BLK
    ).astype(jnp.float32)
    in_used = idx < n_used
    r = jnp.where(in_used, rzt[0:1, :] + br_ref[0, 0], 0.0)
    z = jnp.where(in_used, rzt[1:2, :] + bz_ref[0, 0], _NEG)

    mask = (idx >= starts) & (idx < ends)
    zm = jnp.where(mask, z, _NEG)

    old_max = acc_ref[:, 0:1]
    blk_max = jnp.max(zm, axis=1, keepdims=True)
    new_max = jnp.maximum(old_max, blk_max)
    scale = jnp.exp(old_max - new_max)

    e = jnp.exp(zm - new_max) * mask.astype(jnp.float32)
    s = jnp.sum(e, axis=1, keepdims=True)
    sr = jnp.sum(e * r, axis=1, keepdims=True)

    acc_ref[:, 0:1] = new_max
    acc_ref[:, 1:2] = acc_ref[:, 1:2] * scale + s
    acc_ref[:, 2:3] = acc_ref[:, 2:3] * scale + sr

    @pl.when(i == nprog - 1)
    def _finish():
        denom = acc_ref[:, 1:2]
        preds = acc_ref[:, 2:3] / jnp.where(denom == 0.0, 1.0, denom)
        empty = starts == ends
        preds = jnp.where(empty, _DEFAULT_PRED, preds)
        out_ref[:] = _SCALE * preds


def kernel(x, xlens, W1, b1, Wr, br, Wz, bz):
    n_used = jnp.minimum(jnp.sum(xlens), _TOTAL).astype(jnp.int32)
    nblocks = jnp.clip((n_used + _BLK - 1) // _BLK, 1, _GRID)

    const = lambda i: (0, 0)
    out = pl.pallas_call(
        _fused_kernel,
        grid=(nblocks,),
        in_specs=[
            pl.BlockSpec((_BLK, _D), lambda i: (i, 0)),
            pl.BlockSpec((_H, _D), const),
            pl.BlockSpec((1, _H), const),
            pl.BlockSpec((1, _H), const),
            pl.BlockSpec((1, _H), const),
            pl.BlockSpec((1, 1), const),
            pl.BlockSpec((1, 1), const),
            pl.BlockSpec((1, _B), const),
        ],
        out_specs=pl.BlockSpec((_B, 1), const),
        out_shape=jax.ShapeDtypeStruct((_B, 1), jnp.float32),
        scratch_shapes=[pltpu.VMEM((_B, 8), jnp.float32)],
    )(
        x, W1, Wr, Wz,
        b1.reshape(1, _H), br.reshape(1, 1), bz.reshape(1, 1),
        xlens.reshape(1, _B),
    )
    return out.reshape(_B)
